# Initial kernel scaffold; baseline (speedup 1.0000x reference)
import jax
import jax.numpy as jnp
from jax.experimental import pallas as pl


def kernel(x_left, batch_left, x_right, batch_right, weight):
    raise NotImplementedError("write your pallas kernel here")


# speedup vs baseline: 4.6905x; 4.6905x over previous
import functools

import jax
import jax.numpy as jnp
from jax.experimental import pallas as pl

M = 4096
K = 128
OUT = 64
BM = 256
BN = 256
_F32 = jnp.float32
_HI = jax.lax.Precision.HIGHEST


def _cross_kernel(bd_ref, bs_ref, xd_ref, xs_ref, w_ref, out_ref):
    bd = bd_ref[...]
    bs = bs_ref[...]
    xd = xd_ref[...]
    gmin = bd[0, 0]
    gmax = bd[BM - 1, 0]
    start = jnp.sum((bs < gmin).astype(_F32)).astype(jnp.int32)
    end = jnp.sum((bs <= gmax).astype(_F32)).astype(jnp.int32)
    bn = jnp.int32(BN)
    c0 = jax.lax.div(start, bn)
    c1 = jax.lax.div(end + bn - jnp.int32(1), bn)
    dnorm = jnp.sqrt(jnp.sum(xd * xd, axis=1, keepdims=True))

    def body(c, carry):
        acc, s, cnt = carry
        off = c * jnp.int32(BN)
        xs = xs_ref[pl.ds(off, BN), :]
        bsb = bs_ref[:, pl.ds(off, BN)]
        snorm = jnp.sqrt(jnp.sum(xs * xs, axis=1))[None, :]
        p = jax.lax.dot_general(xd, xs, (((1,), (1,)), ((), ())),
                                precision=_HI)
        nrm = jnp.maximum(dnorm * snorm, 1e-6)
        coef = jnp.maximum(p / nrm, 0.0)
        mask = bd == bsb
        cm = jnp.where(mask, coef, 0.0)
        acc = acc + jax.lax.dot_general(cm, xs, (((1,), (0,)), ((), ())),
                                        precision=_HI)
        s = s + jnp.sum(cm, axis=1, keepdims=True)
        cnt = cnt + jnp.sum(mask.astype(_F32), axis=1, keepdims=True)
        return acc, s, cnt

    acc0 = jnp.zeros((BM, K), _F32)
    s0 = jnp.zeros((BM, 1), _F32)
    acc, s, cnt = jax.lax.fori_loop(c0, c1, body, (acc0, s0, s0))

    denom = s + 1e-6 * cnt
    gx = jnp.where(denom > 0, acc / denom, 0.0)
    w2 = w_ref[...]
    w2 = w2 * w2
    dot = lambda a: jax.lax.dot_general(a, w2, (((1,), (1,)), ((), ())),
                                        precision=_HI)
    num = dot(xd * gx)
    td = jnp.sqrt(dot(xd * xd) + 1e-6)
    gd = jnp.sqrt(dot(gx * gx) + 1e-6)
    out_ref[...] = num / jnp.maximum(td * gd, 1e-6)


def _z():
    return jnp.int32(0)


@functools.partial(jax.jit, static_argnames=("interpret",))
def _run(x_left, bl, x_right, br, weight, interpret=False):
    grid = (M // BM,)
    call = functools.partial(
        pl.pallas_call,
        grid=grid,
        out_shape=jax.ShapeDtypeStruct((M, OUT), _F32),
        in_specs=[
            pl.BlockSpec((BM, 1), lambda i: (i, _z())),
            pl.BlockSpec((1, M), lambda i: (_z(), _z())),
            pl.BlockSpec((BM, K), lambda i: (i, _z())),
            pl.BlockSpec((M, K), lambda i: (_z(), _z())),
            pl.BlockSpec((OUT, K), lambda i: (_z(), _z())),
        ],
        out_specs=pl.BlockSpec((BM, OUT), lambda i: (i, _z())),
        interpret=interpret,
    )
    out1 = call(_cross_kernel)(bl[:, None], br[None, :], x_left, x_right, weight)
    out2 = call(_cross_kernel)(br[:, None], bl[None, :], x_right, x_left, weight)
    return out1, out2


def kernel(x_left, batch_left, x_right, batch_right, weight):
    bl = batch_left.astype(jnp.int32)
    br = batch_right.astype(jnp.int32)
    return _run(x_left, bl, x_right, br, weight)


# speedup vs baseline: 8.1220x; 1.7316x over previous
import functools

import jax
import jax.numpy as jnp
from jax.experimental import pallas as pl
from jax.experimental.pallas import tpu as pltpu

M = 4096
K = 128
OUT = 64
BM = 256
BN = 256
_F32 = jnp.float32
_HI = jax.lax.Precision.HIGHEST
_P1 = jax.lax.Precision.DEFAULT


def _cross_kernel(bd_ref, bs_ref, xd_ref, xs_ref, w_ref, out_ref, xsc_ref):
    @pl.when(pl.program_id(0) == 0)
    def _init():
        xs = xs_ref[...]
        ss = jnp.sum(xs * xs, axis=1, keepdims=True)
        rs = jnp.where(ss < 1e-12, _F32(1e6), jax.lax.rsqrt(ss))
        xsc_ref[...] = xs * rs

    bd = bd_ref[...]
    bs = bs_ref[...]
    xd = xd_ref[...]
    gmin = bd[0, 0]
    gmax = bd[BM - 1, 0]
    start = jnp.sum((bs < gmin).astype(_F32)).astype(jnp.int32)
    end = jnp.sum((bs <= gmax).astype(_F32)).astype(jnp.int32)
    bn = jnp.int32(BN)
    c0 = jax.lax.div(start, bn)
    c1 = jax.lax.div(end + bn - jnp.int32(1), bn)
    dnorm = jnp.sqrt(jnp.sum(xd * xd, axis=1, keepdims=True))
    cden = _F32(1e-6) * dnorm

    def body(c, carry):
        acc, s = carry
        off = c * jnp.int32(BN)
        xsc = xsc_ref[pl.ds(off, BN), :]
        xs = xs_ref[pl.ds(off, BN), :]
        bsb = bs_ref[:, pl.ds(off, BN)]
        p = jax.lax.dot_general(xd, xsc, (((1,), (1,)), ((), ())),
                                precision=_P1)
        q = jnp.maximum(p, 0.0)
        mask = bd == bsb
        cm = jnp.where(mask, q, 0.0)
        w_inc = jnp.where(mask, q + cden, 0.0)
        acc = acc + jax.lax.dot_general(cm, xs, (((1,), (0,)), ((), ())),
                                        precision=_P1)
        s = s + jnp.sum(w_inc, axis=1, keepdims=True)
        return acc, s

    acc0 = jnp.zeros((BM, K), _F32)
    s0 = jnp.zeros((BM, 1), _F32)
    acc, s = jax.lax.fori_loop(c0, c1, body, (acc0, s0))

    gx = jnp.where(s > 0, acc / s, 0.0)
    w2 = w_ref[...]
    w2 = w2 * w2
    dot = lambda a: jax.lax.dot_general(a, w2, (((1,), (1,)), ((), ())),
                                        precision=_HI)
    num = dot(xd * gx)
    td = jnp.sqrt(dot(xd * xd) + 1e-6)
    gd = jnp.sqrt(dot(gx * gx) + 1e-6)
    out_ref[...] = num / jnp.maximum(td * gd, 1e-6)


def _z():
    return jnp.int32(0)


@functools.partial(jax.jit, static_argnames=("interpret",))
def _run(x_left, bl, x_right, br, weight, interpret=False):
    grid = (M // BM,)
    call = functools.partial(
        pl.pallas_call,
        grid=grid,
        out_shape=jax.ShapeDtypeStruct((M, OUT), _F32),
        in_specs=[
            pl.BlockSpec((BM, 1), lambda i: (i, _z())),
            pl.BlockSpec((1, M), lambda i: (_z(), _z())),
            pl.BlockSpec((BM, K), lambda i: (i, _z())),
            pl.BlockSpec((M, K), lambda i: (_z(), _z())),
            pl.BlockSpec((OUT, K), lambda i: (_z(), _z())),
        ],
        out_specs=pl.BlockSpec((BM, OUT), lambda i: (i, _z())),
        scratch_shapes=[pltpu.VMEM((M, K), _F32)],
        interpret=interpret,
    )
    out1 = call(_cross_kernel)(bl[:, None], br[None, :], x_left, x_right, weight)
    out2 = call(_cross_kernel)(br[:, None], bl[None, :], x_right, x_left, weight)
    return out1, out2


def kernel(x_left, batch_left, x_right, batch_right, weight):
    bl = batch_left.astype(jnp.float32)
    br = batch_right.astype(jnp.float32)
    return _run(x_left, bl, x_right, br, weight)


# speedup vs baseline: 8.6085x; 1.0599x over previous
import functools

import jax
import jax.numpy as jnp
from jax.experimental import pallas as pl
from jax.experimental.pallas import tpu as pltpu

M = 4096
K = 128
OUT = 64
BM = 256
BN = 128
_F32 = jnp.float32
_HI = jax.lax.Precision.DEFAULT
_P1 = jax.lax.Precision.DEFAULT


def _cross_kernel(bd_ref, bs_ref, xd_ref, xs_ref, w_ref, out_ref, xsc_ref):
    @pl.when(pl.program_id(0) == 0)
    def _init():
        xs = xs_ref[...]
        ss = jnp.sum(xs * xs, axis=1, keepdims=True)
        rs = jnp.where(ss < 1e-12, _F32(1e6), jax.lax.rsqrt(ss))
        xsc_ref[...] = xs * rs

    bd = bd_ref[...]
    bs = bs_ref[...]
    xd = xd_ref[...]
    gmin = bd[0, 0]
    gmax = bd[BM - 1, 0]
    start = jnp.sum((bs < gmin).astype(_F32)).astype(jnp.int32)
    end = jnp.sum((bs <= gmax).astype(_F32)).astype(jnp.int32)
    bn = jnp.int32(BN)
    c0 = jax.lax.div(start, bn)
    c1 = jax.lax.div(end + bn - jnp.int32(1), bn)
    dnorm = jnp.sqrt(jnp.sum(xd * xd, axis=1, keepdims=True))
    cden = _F32(1e-6) * dnorm

    def body(c, carry):
        acc, s = carry
        off = c * jnp.int32(BN)
        xsc = xsc_ref[pl.ds(off, BN), :]
        xs = xs_ref[pl.ds(off, BN), :]
        bsb = bs_ref[:, pl.ds(off, BN)]
        p = jax.lax.dot_general(xd, xsc, (((1,), (1,)), ((), ())),
                                precision=_P1)
        q = jnp.maximum(p, 0.0)
        mask = bd == bsb
        cm = jnp.where(mask, q, 0.0)
        w_inc = jnp.where(mask, q + cden, 0.0)
        acc = acc + jax.lax.dot_general(cm, xs, (((1,), (0,)), ((), ())),
                                        precision=_P1)
        s = s + jnp.sum(w_inc, axis=1, keepdims=True)
        return acc, s

    acc0 = jnp.zeros((BM, K), _F32)
    s0 = jnp.zeros((BM, 1), _F32)
    acc, s = jax.lax.fori_loop(c0, c1, body, (acc0, s0))

    gx = jnp.where(s > 0, acc / s, 0.0)
    w2 = w_ref[...]
    w2 = w2 * w2
    dot = lambda a: jax.lax.dot_general(a, w2, (((1,), (1,)), ((), ())),
                                        precision=_HI)
    num = dot(xd * gx)
    td = jnp.sqrt(dot(xd * xd) + 1e-6)
    gd = jnp.sqrt(dot(gx * gx) + 1e-6)
    out_ref[...] = num / jnp.maximum(td * gd, 1e-6)


def _z():
    return jnp.int32(0)


@functools.partial(jax.jit, static_argnames=("interpret",))
def _run(x_left, bl, x_right, br, weight, interpret=False):
    grid = (M // BM,)
    call = functools.partial(
        pl.pallas_call,
        grid=grid,
        out_shape=jax.ShapeDtypeStruct((M, OUT), _F32),
        in_specs=[
            pl.BlockSpec((BM, 1), lambda i: (i, _z())),
            pl.BlockSpec((1, M), lambda i: (_z(), _z())),
            pl.BlockSpec((BM, K), lambda i: (i, _z())),
            pl.BlockSpec((M, K), lambda i: (_z(), _z())),
            pl.BlockSpec((OUT, K), lambda i: (_z(), _z())),
        ],
        out_specs=pl.BlockSpec((BM, OUT), lambda i: (i, _z())),
        scratch_shapes=[pltpu.VMEM((M, K), _F32)],
        interpret=interpret,
    )
    out1 = call(_cross_kernel)(bl[:, None], br[None, :], x_left, x_right, weight)
    out2 = call(_cross_kernel)(br[:, None], bl[None, :], x_right, x_left, weight)
    return out1, out2


def kernel(x_left, batch_left, x_right, batch_right, weight):
    bl = batch_left.astype(jnp.float32)
    br = batch_right.astype(jnp.float32)
    return _run(x_left, bl, x_right, br, weight)
